# Initial kernel scaffold; baseline (speedup 1.0000x reference)
import jax
import jax.numpy as jnp
from jax.experimental import pallas as pl


def kernel(x, duty_cycle):
    raise NotImplementedError("write your pallas kernel here")


# speedup vs baseline: 22.5871x; 22.5871x over previous
import jax
import jax.numpy as jnp
from jax import lax
from jax.experimental import pallas as pl

_N = 32768
_B = 128
_K = 3277
_BOOST = 1.0
_ROWS = 8


def _sortable_u32(f):
    bits = lax.bitcast_convert_type(f, jnp.int32)
    m = lax.shift_right_arithmetic(bits, 31)
    enc = bits ^ (m | jnp.int32(-2147483648))
    return lax.bitcast_convert_type(enc, jnp.uint32)


def _tc_body(x_ref, dc_ref, out_ref):
    x = x_ref[...]
    s = jnp.exp(-_BOOST * dc_ref[...])
    b = x * s
    u = _sortable_u32(b)

    def step(i, t):
        bit = lax.shift_right_logical(jnp.uint32(0x80000000),
                                      i.astype(jnp.uint32))
        cand = t | bit
        cnt = jnp.sum((u >= cand).astype(jnp.int32), axis=1, keepdims=True)
        return jnp.where(cnt >= _K, cand, t)

    t = lax.fori_loop(0, 32, step, jnp.zeros((_ROWS, 1), jnp.uint32))
    out_ref[...] = jnp.where(u >= t, x, jnp.float32(0.0))


def kernel(x, duty_cycle):
    dc = duty_cycle.reshape(1, _N)
    return pl.pallas_call(
        _tc_body,
        grid=(_B // _ROWS,),
        in_specs=[
            pl.BlockSpec((_ROWS, _N), lambda i: (i, 0)),
            pl.BlockSpec((1, _N), lambda i: (0, 0)),
        ],
        out_specs=pl.BlockSpec((_ROWS, _N), lambda i: (i, 0)),
        out_shape=jax.ShapeDtypeStruct((_B, _N), jnp.float32),
    )(x, dc)
